# Initial kernel scaffold; baseline (speedup 1.0000x reference)
import jax
import jax.numpy as jnp
from jax.experimental import pallas as pl


def kernel(x, Wconv, Wgcn, bgcn):
    raise NotImplementedError("write your pallas kernel here")


# speedup vs baseline: 9.4996x; 9.4996x over previous
import jax
import jax.numpy as jnp
import numpy as np
from jax import lax
from jax.experimental import pallas as pl

_P = 2
_C = 96
_IMG = 512
_W = _IMG // _P
_H = _IMG // _P
_N = _H * _W
_NB = 4096
_NSTEPS = _N // _NB
_HALO = _W
_STRAY_DST = _N - 1 - _W - 1
_STRAY_SRC = _N - 1


def _body(prev_ref, cur_ref, next_ref, wf_ref, wg_ref, b_ref, out_ref):
    i = pl.program_id(0)
    wcomb = jnp.dot(wf_ref[...], wg_ref[...], preferred_element_type=jnp.float32)

    xp = jnp.concatenate(
        [prev_ref[_NB - _HALO:, :], cur_ref[...], next_ref[:_HALO, :]], axis=0)

    xw = (xp[:, 0:1] * wcomb[0:1, :] + xp[:, 1:2] * wcomb[1:2, :]
          + xp[:, 2:3] * wcomb[2:3, :] + xp[:, 3:4] * wcomb[3:4, :])

    nh = _NB + 2 * _HALO
    gh = i * _NB - _HALO + lax.broadcasted_iota(jnp.int32, (nh, 1), 0)
    row = jnp.floor_divide(gh, _W)
    col = gh - row * _W
    deg = (1.0
           + (row > 0).astype(jnp.float32)
           + (row < _H - 1).astype(jnp.float32)
           + (col > 0).astype(jnp.float32)
           + (col < _W - 1).astype(jnp.float32)
           + (gh == _STRAY_DST).astype(jnp.float32))
    dinv = lax.rsqrt(deg)
    z = xw * dinv

    g = i * _NB + lax.broadcasted_iota(jnp.int32, (_NB, 1), 0)
    colc = jnp.remainder(g, _W)
    z_self = z[_HALO:_HALO + _NB]
    z_up = z[0:_NB]
    z_dn = z[2 * _HALO:2 * _HALO + _NB]
    z_lt = jnp.where(colc != 0, z[_HALO - 1:_HALO - 1 + _NB], 0.0)
    z_rt = jnp.where(colc != _W - 1, z[_HALO + 1:_HALO + 1 + _NB], 0.0)
    zsum = z_self + z_up + z_dn + z_lt + z_rt

    out_ref[...] = zsum * dinv[_HALO:_HALO + _NB] + b_ref[...]

    @pl.when(i == _NSTEPS - 1)
    def _():
        base = (_NSTEPS - 1) * _NB
        loc = _STRAY_DST - base
        off = _STRAY_SRC - base + _HALO
        corr = np.float32(1.0 / np.sqrt(6.0)) * z[off:off + 1, :]
        out_ref[loc:loc + 1, :] = out_ref[loc:loc + 1, :] + corr


def kernel(x, Wconv, Wgcn, bgcn):
    p = x.reshape(_IMG, _IMG)
    xpatch = p.reshape(_H, _P, _W, _P).transpose(0, 2, 1, 3).reshape(_N, _P * _P)
    xpad = jnp.pad(xpatch, ((_NB, _NB), (0, 0)))

    wf = Wconv.reshape(_C, _P * _P).T.astype(jnp.float32)
    wg = Wgcn.T.astype(jnp.float32)
    b = bgcn.reshape(1, _C).astype(jnp.float32)

    out = pl.pallas_call(
        _body,
        grid=(_NSTEPS,),
        in_specs=[
            pl.BlockSpec((_NB, _P * _P), lambda i: (i, 0)),
            pl.BlockSpec((_NB, _P * _P), lambda i: (i + 1, 0)),
            pl.BlockSpec((_NB, _P * _P), lambda i: (i + 2, 0)),
            pl.BlockSpec((_P * _P, _C), lambda i: (0, 0)),
            pl.BlockSpec((_C, _C), lambda i: (0, 0)),
            pl.BlockSpec((1, _C), lambda i: (0, 0)),
        ],
        out_specs=pl.BlockSpec((_NB, _C), lambda i: (i, 0)),
        out_shape=jax.ShapeDtypeStruct((_N, _C), jnp.float32),
    )(xpad, xpad, xpad, wf, wg, b)

    return out.reshape(1, _N, _C)


# speedup vs baseline: 23.9985x; 2.5263x over previous
import jax
import jax.numpy as jnp
import numpy as np
from jax import lax
from jax.experimental import pallas as pl

_P = 2
_C = 96
_IMG = 512
_W = _IMG // _P
_H = _IMG // _P
_N = _H * _W
_R = 16
_NSTEPS = _H // _R
_WP = _W + 2
_STRAY_R = _H - 2
_STRAY_C = _W - 2


def _build_perm() -> np.ndarray:
    s = np.zeros((2 * _IMG, 4 * _WP), dtype=np.float32)
    for kh in range(_P):
        for kw in range(_P):
            k = _P * kh + kw
            for c in range(_W):
                s[_IMG * kh + _P * c + kw, k * _WP + c + 1] = 1.0
    return s


def _build_dinv() -> np.ndarray:
    r = np.arange(-1, _H + 1)
    c = np.arange(_W)
    deg = (1.0 + (r[None, :] > 0) + (r[None, :] < _H - 1)
           + (c[:, None] > 0) + (c[:, None] < _W - 1)
           + ((r[None, :] == _STRAY_R) & (c[:, None] == _STRAY_C)))
    dinv = 1.0 / np.sqrt(deg)
    out = np.zeros((_NSTEPS, _WP, _R + 2), dtype=np.float32)
    for i in range(_NSTEPS):
        out[i, 1:_W + 1, :] = dinv[:, i * _R: i * _R + _R + 2]
    return out


_SM = _build_perm()
_DINV = _build_dinv()


def _body(xp_ref, sm_ref, wf_ref, wg_ref, b_ref, dv_ref, out_ref):
    i = pl.program_id(0)
    wcomb = jnp.dot(wf_ref[...], wg_ref[...], preferred_element_type=jnp.float32)
    bias = b_ref[...]
    dblk = dv_ref[0]

    xh = xp_ref[pl.ds(i * _R, _R + 2), :]
    ut = lax.dot_general(sm_ref[...], xh, (((0,), (1,)), ((), ())),
                         preferred_element_type=jnp.float32)
    uk = [ut[k * _WP:(k + 1) * _WP, :] for k in range(4)]

    def mk_z(rl):
        acc = (uk[0][:, rl:rl + 1] * wcomb[0:1, :]
               + uk[1][:, rl:rl + 1] * wcomb[1:2, :]
               + uk[2][:, rl:rl + 1] * wcomb[2:3, :]
               + uk[3][:, rl:rl + 1] * wcomb[3:4, :])
        return acc * dblk[:, rl:rl + 1]

    zm = mk_z(0)
    zc = mk_z(1)
    for r in range(_R):
        zn = mk_z(r + 2)
        osum = (zm[1:_W + 1] + zc[1:_W + 1] + zn[1:_W + 1]
                + zc[0:_W] + zc[2:_W + 2])
        orow = osum * dblk[1:_W + 1, r + 1:r + 2] + bias
        out_ref[r * _W:(r + 1) * _W, :] = orow
        if r == _STRAY_R % _R:
            @pl.when(i == _NSTEPS - 1)
            def _():
                loc = (_STRAY_R % _R) * _W + _STRAY_C
                corr = np.float32(1.0 / np.sqrt(6.0)) * zn[_W:_W + 1, :]
                out_ref[loc:loc + 1, :] = out_ref[loc:loc + 1, :] + corr
        zm, zc = zc, zn


def kernel(x, Wconv, Wgcn, bgcn):
    xr = x.reshape(_H, 2 * _IMG)
    xpad = jnp.pad(xr, ((1, 1), (0, 0)))
    wf = Wconv.reshape(_C, 4).T.astype(jnp.float32)
    wg = Wgcn.T.astype(jnp.float32)
    b = bgcn.reshape(1, _C).astype(jnp.float32)
    sm = jnp.asarray(_SM)
    dv = jnp.asarray(_DINV)

    out = pl.pallas_call(
        _body,
        grid=(_NSTEPS,),
        in_specs=[
            pl.BlockSpec((_H + 2, 2 * _IMG), lambda i: (0, 0)),
            pl.BlockSpec((2 * _IMG, 4 * _WP), lambda i: (0, 0)),
            pl.BlockSpec((4, _C), lambda i: (0, 0)),
            pl.BlockSpec((_C, _C), lambda i: (0, 0)),
            pl.BlockSpec((1, _C), lambda i: (0, 0)),
            pl.BlockSpec((1, _WP, _R + 2), lambda i: (i, 0, 0)),
        ],
        out_specs=pl.BlockSpec((_R * _W, _C), lambda i: (i, 0)),
        out_shape=jax.ShapeDtypeStruct((_N, _C), jnp.float32),
    )(xpad, sm, wf, wg, b, dv)

    return out.reshape(1, _N, _C)


# speedup vs baseline: 28.9215x; 1.2051x over previous
import jax
import jax.numpy as jnp
import numpy as np
from jax.experimental import pallas as pl

_P = 2
_C = 96
_IMG = 512
_W = _IMG // _P
_H = _IMG // _P
_N = _H * _W
_R = 16
_NSTEPS = _H // _R


def _dinv_grid() -> np.ndarray:
    r = np.arange(_H)[:, None]
    c = np.arange(_W)[None, :]
    deg = (1.0 + (r > 0) + (r < _H - 1) + (c > 0) + (c < _W - 1)
           + ((r == _H - 2) & (c == _W - 2)))
    return (1.0 / np.sqrt(deg)).astype(np.float32)


def _build_perm() -> np.ndarray:
    s = np.zeros((2 * _IMG, 4 * _W), dtype=np.float32)
    for kh in range(_P):
        for kw in range(_P):
            k = _P * kh + kw
            for c in range(_W):
                s[_IMG * kh + _P * c + kw, k * _W + c] = 1.0
    return s


def _build_dsrc() -> np.ndarray:
    dinv = _dinv_grid()
    out = np.zeros((_NSTEPS, _R + 2, 4 * _W), dtype=np.float32)
    for i in range(_NSTEPS):
        for rl in range(_R + 2):
            r = i * _R + rl - 1
            if 0 <= r < _H:
                out[i, rl, :] = np.tile(dinv[r], 4)
    return out


def _build_lane_masks():
    ml = np.ones((1, 4 * _W), dtype=np.float32)
    mr = np.ones((1, 4 * _W), dtype=np.float32)
    for k in range(4):
        ml[0, k * _W] = 0.0
        mr[0, (k + 1) * _W - 1] = 0.0
    return ml, mr


_SM = _build_perm()
_DSRC = _build_dsrc()
_ML, _MR = _build_lane_masks()
_DDST = _dinv_grid().reshape(_NSTEPS, _R * _W, 1)


def _body(xp_ref, sm_ref, wf_ref, wg_ref, b_ref, dv_ref, ml_ref, mr_ref,
          dd_ref, out_ref):
    i = pl.program_id(0)
    wcomb = jnp.dot(wf_ref[...], wg_ref[...], preferred_element_type=jnp.float32)

    ra = xp_ref[pl.ds(jnp.maximum(i * _R - 1, 0), 1), :]
    rm = xp_ref[pl.ds(i * _R, _R), :]
    rb = xp_ref[pl.ds(jnp.minimum(i * _R + _R, _H - 1), 1), :]
    xh = jnp.concatenate([ra, rm, rb], axis=0)

    ut = jnp.dot(xh, sm_ref[...], preferred_element_type=jnp.float32)
    ut = ut * dv_ref[0]

    ct = ut[1:_R + 1]
    zc1 = jnp.zeros((_R, 1), jnp.float32)
    shl = jnp.concatenate([zc1, ct[:, :4 * _W - 1]], axis=1) * ml_ref[...]
    shr = jnp.concatenate([ct[:, 1:], zc1], axis=1) * mr_ref[...]
    usum = ut[0:_R] + ct + ut[2:_R + 2] + shl + shr

    ust = usum.T
    prows = []
    for r in range(_R):
        prows.append(jnp.concatenate(
            [ust[k * _W:(k + 1) * _W, r:r + 1] for k in range(4)], axis=1))
    p = jnp.concatenate(prows, axis=0)
    z = jnp.dot(p, wcomb, preferred_element_type=jnp.float32)
    out_ref[...] = z * dd_ref[0] + b_ref[...]

    @pl.when(i == _NSTEPS - 1)
    def _():
        loc = 14 * _W + (_W - 2)
        src = jnp.concatenate(
            [ut[_R:_R + 1, (k + 1) * _W - 1:(k + 1) * _W]
             for k in range(4)], axis=1)
        corr = np.float32(1.0 / np.sqrt(6.0)) * jnp.dot(
            src, wcomb, preferred_element_type=jnp.float32)
        out_ref[loc:loc + 1, :] = out_ref[loc:loc + 1, :] + corr


def kernel(x, Wconv, Wgcn, bgcn):
    xr = x.reshape(_H, 2 * _IMG)
    wf = Wconv.reshape(_C, 4).T.astype(jnp.float32)
    wg = Wgcn.T.astype(jnp.float32)
    b = bgcn.reshape(1, _C).astype(jnp.float32)

    out = pl.pallas_call(
        _body,
        grid=(_NSTEPS,),
        in_specs=[
            pl.BlockSpec((_H, 2 * _IMG), lambda i: (0, 0)),
            pl.BlockSpec((2 * _IMG, 4 * _W), lambda i: (0, 0)),
            pl.BlockSpec((4, _C), lambda i: (0, 0)),
            pl.BlockSpec((_C, _C), lambda i: (0, 0)),
            pl.BlockSpec((1, _C), lambda i: (0, 0)),
            pl.BlockSpec((1, _R + 2, 4 * _W), lambda i: (i, 0, 0)),
            pl.BlockSpec((1, 4 * _W), lambda i: (0, 0)),
            pl.BlockSpec((1, 4 * _W), lambda i: (0, 0)),
            pl.BlockSpec((1, _R * _W, 1), lambda i: (i, 0, 0)),
        ],
        out_specs=pl.BlockSpec((_R * _W, _C), lambda i: (i, 0)),
        out_shape=jax.ShapeDtypeStruct((_N, _C), jnp.float32),
    )(xr, jnp.asarray(_SM), wf, wg, b, jnp.asarray(_DSRC),
      jnp.asarray(_ML), jnp.asarray(_MR), jnp.asarray(_DDST))

    return out.reshape(1, _N, _C)


# speedup vs baseline: 29.6019x; 1.0235x over previous
import jax
import jax.numpy as jnp
import numpy as np
from jax.experimental import pallas as pl

_P = 2
_C = 96
_IMG = 512
_W = _IMG // _P
_H = _IMG // _P
_N = _H * _W
_R = 16
_NSTEPS = _H // _R


def _dinv_grid() -> np.ndarray:
    r = np.arange(_H)[:, None]
    c = np.arange(_W)[None, :]
    deg = (1.0 + (r > 0) + (r < _H - 1) + (c > 0) + (c < _W - 1)
           + ((r == _H - 2) & (c == _W - 2)))
    return (1.0 / np.sqrt(deg)).astype(np.float32)


def _build_perm() -> np.ndarray:
    s = np.zeros((2 * _IMG, 4 * _W), dtype=np.float32)
    for kh in range(_P):
        for kw in range(_P):
            k = _P * kh + kw
            for c in range(_W):
                s[_IMG * kh + _P * c + kw, k * _W + c] = 1.0
    return s


def _build_dsrc() -> np.ndarray:
    dinv = _dinv_grid()
    out = np.zeros((_NSTEPS, _R + 2, 4 * _W), dtype=np.float32)
    for i in range(_NSTEPS):
        for rl in range(_R + 2):
            r = i * _R + rl - 1
            if 0 <= r < _H:
                out[i, rl, :] = np.tile(dinv[r], 4)
    return out


def _build_lane_masks():
    ml = np.ones((1, 4 * _W), dtype=np.float32)
    mr = np.ones((1, 4 * _W), dtype=np.float32)
    for k in range(4):
        ml[0, k * _W] = 0.0
        mr[0, (k + 1) * _W - 1] = 0.0
    return ml, mr


_SM = _build_perm()
_DSRC = _build_dsrc()
_ML, _MR = _build_lane_masks()
_DDST = _dinv_grid().reshape(_NSTEPS, _R * _W, 1)


def _body(xpv_ref, xcv_ref, xnv_ref, sm_ref, wf_ref, wg_ref, b_ref, dv_ref,
          ml_ref, mr_ref, dd_ref, out_ref):
    i = pl.program_id(0)
    wcomb = jnp.dot(wf_ref[...], wg_ref[...], preferred_element_type=jnp.float32)

    xb = jnp.concatenate(
        [xpv_ref[2 * _R - 2:, :], xcv_ref[...], xnv_ref[:2, :]],
        axis=0)
    xh = jnp.concatenate(
        [jnp.concatenate([xb[2 * rl:2 * rl + 1, :],
                          xb[2 * rl + 1:2 * rl + 2, :]], axis=1)
         for rl in range(_R + 2)], axis=0)

    ut = jnp.dot(xh, sm_ref[...], preferred_element_type=jnp.float32)
    ut = ut * dv_ref[0]

    ct = ut[1:_R + 1]
    zc1 = jnp.zeros((_R, 1), jnp.float32)
    shl = jnp.concatenate([zc1, ct[:, :4 * _W - 1]], axis=1) * ml_ref[...]
    shr = jnp.concatenate([ct[:, 1:], zc1], axis=1) * mr_ref[...]
    usum = ut[0:_R] + ct + ut[2:_R + 2] + shl + shr

    ust = usum.T
    prows = []
    for r in range(_R):
        prows.append(jnp.concatenate(
            [ust[k * _W:(k + 1) * _W, r:r + 1] for k in range(4)], axis=1))
    p = jnp.concatenate(prows, axis=0)
    z = jnp.dot(p, wcomb, preferred_element_type=jnp.float32)
    out_ref[...] = z * dd_ref[0] + b_ref[...]

    @pl.when(i == _NSTEPS - 1)
    def _():
        loc = 14 * _W + (_W - 2)
        src = jnp.concatenate(
            [ut[_R:_R + 1, (k + 1) * _W - 1:(k + 1) * _W]
             for k in range(4)], axis=1)
        corr = np.float32(1.0 / np.sqrt(6.0)) * jnp.dot(
            src, wcomb, preferred_element_type=jnp.float32)
        out_ref[loc:loc + 1, :] = out_ref[loc:loc + 1, :] + corr


def kernel(x, Wconv, Wgcn, bgcn):
    xr = x.reshape(_IMG, _IMG)
    wf = Wconv.reshape(_C, 4).T.astype(jnp.float32)
    wg = Wgcn.T.astype(jnp.float32)
    b = bgcn.reshape(1, _C).astype(jnp.float32)

    out = pl.pallas_call(
        _body,
        grid=(_NSTEPS,),
        in_specs=[
            pl.BlockSpec((2 * _R, _IMG), lambda i: (jnp.maximum(i - 1, 0), 0)),
            pl.BlockSpec((2 * _R, _IMG), lambda i: (i, 0)),
            pl.BlockSpec((2 * _R, _IMG),
                         lambda i: (jnp.minimum(i + 1, _NSTEPS - 1), 0)),
            pl.BlockSpec((2 * _IMG, 4 * _W), lambda i: (0, 0)),
            pl.BlockSpec((4, _C), lambda i: (0, 0)),
            pl.BlockSpec((_C, _C), lambda i: (0, 0)),
            pl.BlockSpec((1, _C), lambda i: (0, 0)),
            pl.BlockSpec((1, _R + 2, 4 * _W), lambda i: (i, 0, 0)),
            pl.BlockSpec((1, 4 * _W), lambda i: (0, 0)),
            pl.BlockSpec((1, 4 * _W), lambda i: (0, 0)),
            pl.BlockSpec((1, _R * _W, 1), lambda i: (i, 0, 0)),
        ],
        out_specs=pl.BlockSpec((_R * _W, _C), lambda i: (i, 0)),
        out_shape=jax.ShapeDtypeStruct((_N, _C), jnp.float32),
    )(xr, xr, xr, jnp.asarray(_SM), wf, wg, b, jnp.asarray(_DSRC),
      jnp.asarray(_ML), jnp.asarray(_MR), jnp.asarray(_DDST))

    return out.reshape(1, _N, _C)


# speedup vs baseline: 116.3585x; 3.9308x over previous
import jax
import jax.numpy as jnp
import numpy as np
from jax.experimental import pallas as pl

_P = 2
_C = 96
_IMG = 512
_W = _IMG // _P
_H = _IMG // _P
_N = _H * _W
_R = 16
_NSTEPS = _H // _R


def _dinv_grid() -> np.ndarray:
    r = np.arange(_H)[:, None]
    c = np.arange(_W)[None, :]
    deg = (1.0 + (r > 0) + (r < _H - 1) + (c > 0) + (c < _W - 1)
           + ((r == _H - 2) & (c == _W - 2)))
    return (1.0 / np.sqrt(deg)).astype(np.float32)


def _build_perm() -> np.ndarray:
    s = np.zeros((2 * _IMG, 4 * _W), dtype=np.float32)
    for kh in range(_P):
        for kw in range(_P):
            k = _P * kh + kw
            for c in range(_W):
                s[_IMG * kh + _P * c + kw, k * _W + c] = 1.0
    return s


def _build_dsrc() -> np.ndarray:
    dinv = _dinv_grid()
    out = np.zeros((_NSTEPS, _R + 2, 4 * _W), dtype=np.float32)
    for i in range(_NSTEPS):
        for rl in range(_R + 2):
            r = i * _R + rl - 1
            if 0 <= r < _H:
                out[i, rl, :] = np.tile(dinv[r], 4)
    return out


def _build_lane_masks():
    ml = np.ones((1, 4 * _W), dtype=np.float32)
    mr = np.ones((1, 4 * _W), dtype=np.float32)
    for k in range(4):
        ml[0, k * _W] = 0.0
        mr[0, (k + 1) * _W - 1] = 0.0
    return ml, mr


_SM = _build_perm()
_DSRC = _build_dsrc()
_ML, _MR = _build_lane_masks()
_DDST = _dinv_grid().reshape(_NSTEPS, 1, _R * _W)


def _body(xpv_ref, xcv_ref, xnv_ref, sm_ref, wf_ref, wg_ref, b_ref, dv_ref,
          ml_ref, mr_ref, dd_ref, out_ref):
    i = pl.program_id(0)
    wct = jnp.dot(wg_ref[...], wf_ref[...], preferred_element_type=jnp.float32)

    xb = jnp.concatenate(
        [xpv_ref[2 * _R - 2:, :], xcv_ref[...], xnv_ref[:2, :]],
        axis=0)
    xh = jnp.concatenate(
        [jnp.concatenate([xb[2 * rl:2 * rl + 1, :],
                          xb[2 * rl + 1:2 * rl + 2, :]], axis=1)
         for rl in range(_R + 2)], axis=0)

    ut = jnp.dot(xh, sm_ref[...], preferred_element_type=jnp.float32)
    ut = ut * dv_ref[0]

    ct = ut[1:_R + 1]
    zc1 = jnp.zeros((_R, 1), jnp.float32)
    shl = jnp.concatenate([zc1, ct[:, :4 * _W - 1]], axis=1) * ml_ref[...]
    shr = jnp.concatenate([ct[:, 1:], zc1], axis=1) * mr_ref[...]
    usum = ut[0:_R] + ct + ut[2:_R + 2] + shl + shr

    pt = jnp.concatenate(
        [jnp.concatenate([usum[r:r + 1, k * _W:(k + 1) * _W]
                          for r in range(_R)], axis=1)
         for k in range(4)], axis=0)
    z = jnp.dot(wct, pt, preferred_element_type=jnp.float32)
    out_ref[...] = z * dd_ref[0] + b_ref[...]

    @pl.when(i == _NSTEPS - 1)
    def _():
        loc = 14 * _W + (_W - 2)
        src = jnp.concatenate(
            [ut[_R:_R + 1, (k + 1) * _W - 1:(k + 1) * _W]
             for k in range(4)], axis=0)
        corr = np.float32(1.0 / np.sqrt(6.0)) * jnp.dot(
            wct, src, preferred_element_type=jnp.float32)
        out_ref[:, loc:loc + 1] = out_ref[:, loc:loc + 1] + corr


def kernel(x, Wconv, Wgcn, bgcn):
    xr = x.reshape(_IMG, _IMG)
    wf = Wconv.reshape(_C, 4).astype(jnp.float32)
    wg = Wgcn.astype(jnp.float32)
    b = bgcn.reshape(_C, 1).astype(jnp.float32)

    out = pl.pallas_call(
        _body,
        grid=(_NSTEPS,),
        in_specs=[
            pl.BlockSpec((2 * _R, _IMG), lambda i: (jnp.maximum(i - 1, 0), 0)),
            pl.BlockSpec((2 * _R, _IMG), lambda i: (i, 0)),
            pl.BlockSpec((2 * _R, _IMG),
                         lambda i: (jnp.minimum(i + 1, _NSTEPS - 1), 0)),
            pl.BlockSpec((2 * _IMG, 4 * _W), lambda i: (0, 0)),
            pl.BlockSpec((_C, 4), lambda i: (0, 0)),
            pl.BlockSpec((_C, _C), lambda i: (0, 0)),
            pl.BlockSpec((_C, 1), lambda i: (0, 0)),
            pl.BlockSpec((1, _R + 2, 4 * _W), lambda i: (i, 0, 0)),
            pl.BlockSpec((1, 4 * _W), lambda i: (0, 0)),
            pl.BlockSpec((1, 4 * _W), lambda i: (0, 0)),
            pl.BlockSpec((1, 1, _R * _W), lambda i: (i, 0, 0)),
        ],
        out_specs=pl.BlockSpec((_C, _R * _W), lambda i: (0, i)),
        out_shape=jax.ShapeDtypeStruct((_C, _N), jnp.float32),
    )(xr, xr, xr, jnp.asarray(_SM), wf, wg, b, jnp.asarray(_DSRC),
      jnp.asarray(_ML), jnp.asarray(_MR), jnp.asarray(_DDST))

    return out.T.reshape(1, _N, _C)


# speedup vs baseline: 152.1195x; 1.3073x over previous
import jax
import jax.numpy as jnp
import numpy as np
from jax.experimental import pallas as pl

_P = 2
_C = 96
_IMG = 512
_W = _IMG // _P
_H = _IMG // _P
_N = _H * _W
_R = 32
_NSTEPS = _H // _R


def _dinv_grid() -> np.ndarray:
    r = np.arange(_H)[:, None]
    c = np.arange(_W)[None, :]
    deg = (1.0 + (r > 0) + (r < _H - 1) + (c > 0) + (c < _W - 1)
           + ((r == _H - 2) & (c == _W - 2)))
    return (1.0 / np.sqrt(deg)).astype(np.float32)


def _build_perm() -> np.ndarray:
    s = np.zeros((2 * _IMG, 4 * _W), dtype=np.float32)
    for kh in range(_P):
        for kw in range(_P):
            k = _P * kh + kw
            for c in range(_W):
                s[_IMG * kh + _P * c + kw, k * _W + c] = 1.0
    return s


def _build_dsrc() -> np.ndarray:
    dinv = _dinv_grid()
    out = np.zeros((_NSTEPS, _R + 2, 4 * _W), dtype=np.float32)
    for i in range(_NSTEPS):
        for rl in range(_R + 2):
            r = i * _R + rl - 1
            if 0 <= r < _H:
                out[i, rl, :] = np.tile(dinv[r], 4)
    return out


def _build_lane_masks():
    ml = np.ones((1, 4 * _W), dtype=np.float32)
    mr = np.ones((1, 4 * _W), dtype=np.float32)
    for k in range(4):
        ml[0, k * _W] = 0.0
        mr[0, (k + 1) * _W - 1] = 0.0
    return ml, mr


_SM = _build_perm()
_DSRC = _build_dsrc()
_ML, _MR = _build_lane_masks()
_DDST = _dinv_grid().reshape(_NSTEPS, 1, _R * _W)


def _body(xpv_ref, xcv_ref, xnv_ref, sm_ref, wf_ref, wg_ref, b_ref, dv_ref,
          ml_ref, mr_ref, dd_ref, out_ref):
    i = pl.program_id(0)
    wct = jnp.dot(wg_ref[...], wf_ref[...], preferred_element_type=jnp.float32)

    xb = jnp.concatenate(
        [xpv_ref[2 * _R - 2:, :], xcv_ref[...], xnv_ref[:2, :]],
        axis=0)
    xh = jnp.concatenate(
        [jnp.concatenate([xb[2 * rl:2 * rl + 1, :],
                          xb[2 * rl + 1:2 * rl + 2, :]], axis=1)
         for rl in range(_R + 2)], axis=0)

    ut = jnp.dot(xh, sm_ref[...], preferred_element_type=jnp.float32)
    ut = ut * dv_ref[0]

    ct = ut[1:_R + 1]
    zc1 = jnp.zeros((_R, 1), jnp.float32)
    shl = jnp.concatenate([zc1, ct[:, :4 * _W - 1]], axis=1) * ml_ref[...]
    shr = jnp.concatenate([ct[:, 1:], zc1], axis=1) * mr_ref[...]
    usum = ut[0:_R] + ct + ut[2:_R + 2] + shl + shr

    pt = jnp.concatenate(
        [jnp.concatenate([usum[r:r + 1, k * _W:(k + 1) * _W]
                          for r in range(_R)], axis=1)
         for k in range(4)], axis=0)
    z = jnp.dot(wct, pt, preferred_element_type=jnp.float32)
    out_ref[...] = z * dd_ref[0] + b_ref[...]

    @pl.when(i == _NSTEPS - 1)
    def _():
        loc = ((_H - 2) % _R) * _W + (_W - 2)
        src = jnp.concatenate(
            [ut[_R:_R + 1, (k + 1) * _W - 1:(k + 1) * _W]
             for k in range(4)], axis=0)
        corr = np.float32(1.0 / np.sqrt(6.0)) * jnp.dot(
            wct, src, preferred_element_type=jnp.float32)
        out_ref[:, loc:loc + 1] = out_ref[:, loc:loc + 1] + corr


def kernel(x, Wconv, Wgcn, bgcn):
    xr = x.reshape(_IMG, _IMG)
    wf = Wconv.reshape(_C, 4).astype(jnp.float32)
    wg = Wgcn.astype(jnp.float32)
    b = bgcn.reshape(_C, 1).astype(jnp.float32)

    out = pl.pallas_call(
        _body,
        grid=(_NSTEPS,),
        in_specs=[
            pl.BlockSpec((2 * _R, _IMG), lambda i: (jnp.maximum(i - 1, 0), 0)),
            pl.BlockSpec((2 * _R, _IMG), lambda i: (i, 0)),
            pl.BlockSpec((2 * _R, _IMG),
                         lambda i: (jnp.minimum(i + 1, _NSTEPS - 1), 0)),
            pl.BlockSpec((2 * _IMG, 4 * _W), lambda i: (0, 0)),
            pl.BlockSpec((_C, 4), lambda i: (0, 0)),
            pl.BlockSpec((_C, _C), lambda i: (0, 0)),
            pl.BlockSpec((_C, 1), lambda i: (0, 0)),
            pl.BlockSpec((1, _R + 2, 4 * _W), lambda i: (i, 0, 0)),
            pl.BlockSpec((1, 4 * _W), lambda i: (0, 0)),
            pl.BlockSpec((1, 4 * _W), lambda i: (0, 0)),
            pl.BlockSpec((1, 1, _R * _W), lambda i: (i, 0, 0)),
        ],
        out_specs=pl.BlockSpec((_C, _R * _W), lambda i: (0, i)),
        out_shape=jax.ShapeDtypeStruct((_C, _N), jnp.float32),
    )(xr, xr, xr, jnp.asarray(_SM), wf, wg, b, jnp.asarray(_DSRC),
      jnp.asarray(_ML), jnp.asarray(_MR), jnp.asarray(_DDST))

    return out.T.reshape(1, _N, _C)


# speedup vs baseline: 171.2325x; 1.1256x over previous
import jax
import jax.numpy as jnp
import numpy as np
from jax.experimental import pallas as pl

_P = 2
_C = 96
_IMG = 512
_W = _IMG // _P
_H = _IMG // _P
_N = _H * _W
_R = 64
_NSTEPS = _H // _R


def _dinv_grid() -> np.ndarray:
    r = np.arange(_H)[:, None]
    c = np.arange(_W)[None, :]
    deg = (1.0 + (r > 0) + (r < _H - 1) + (c > 0) + (c < _W - 1)
           + ((r == _H - 2) & (c == _W - 2)))
    return (1.0 / np.sqrt(deg)).astype(np.float32)


def _build_perm() -> np.ndarray:
    s = np.zeros((2 * _IMG, 4 * _W), dtype=np.float32)
    for kh in range(_P):
        for kw in range(_P):
            k = _P * kh + kw
            for c in range(_W):
                s[_IMG * kh + _P * c + kw, k * _W + c] = 1.0
    return s


def _build_dsrc() -> np.ndarray:
    dinv = _dinv_grid()
    out = np.zeros((_NSTEPS, _R + 2, 4 * _W), dtype=np.float32)
    for i in range(_NSTEPS):
        for rl in range(_R + 2):
            r = i * _R + rl - 1
            if 0 <= r < _H:
                out[i, rl, :] = np.tile(dinv[r], 4)
    return out


def _build_lane_masks():
    ml = np.ones((1, 4 * _W), dtype=np.float32)
    mr = np.ones((1, 4 * _W), dtype=np.float32)
    for k in range(4):
        ml[0, k * _W] = 0.0
        mr[0, (k + 1) * _W - 1] = 0.0
    return ml, mr


_SM = _build_perm()
_DSRC = _build_dsrc()
_ML, _MR = _build_lane_masks()
_DDST = _dinv_grid().reshape(_NSTEPS, 1, _R * _W)


def _body(xpv_ref, xcv_ref, xnv_ref, sm_ref, wf_ref, wg_ref, b_ref, dv_ref,
          ml_ref, mr_ref, dd_ref, out_ref):
    i = pl.program_id(0)
    wct = jnp.dot(wg_ref[...], wf_ref[...], preferred_element_type=jnp.float32)

    xb = jnp.concatenate(
        [xpv_ref[2 * _R - 2:, :], xcv_ref[...], xnv_ref[:2, :]],
        axis=0)
    xh = jnp.concatenate(
        [jnp.concatenate([xb[2 * rl:2 * rl + 1, :],
                          xb[2 * rl + 1:2 * rl + 2, :]], axis=1)
         for rl in range(_R + 2)], axis=0)

    ut = jnp.dot(xh, sm_ref[...], preferred_element_type=jnp.float32)
    ut = ut * dv_ref[0]

    ct = ut[1:_R + 1]
    zc1 = jnp.zeros((_R, 1), jnp.float32)
    shl = jnp.concatenate([zc1, ct[:, :4 * _W - 1]], axis=1) * ml_ref[...]
    shr = jnp.concatenate([ct[:, 1:], zc1], axis=1) * mr_ref[...]
    usum = ut[0:_R] + ct + ut[2:_R + 2] + shl + shr

    pt = jnp.concatenate(
        [jnp.concatenate([usum[r:r + 1, k * _W:(k + 1) * _W]
                          for r in range(_R)], axis=1)
         for k in range(4)], axis=0)
    z = jnp.dot(wct, pt, preferred_element_type=jnp.float32)
    out_ref[...] = z * dd_ref[0] + b_ref[...]

    @pl.when(i == _NSTEPS - 1)
    def _():
        loc = ((_H - 2) % _R) * _W + (_W - 2)
        src = jnp.concatenate(
            [ut[_R:_R + 1, (k + 1) * _W - 1:(k + 1) * _W]
             for k in range(4)], axis=0)
        corr = np.float32(1.0 / np.sqrt(6.0)) * jnp.dot(
            wct, src, preferred_element_type=jnp.float32)
        out_ref[:, loc:loc + 1] = out_ref[:, loc:loc + 1] + corr


def kernel(x, Wconv, Wgcn, bgcn):
    xr = x.reshape(_IMG, _IMG)
    wf = Wconv.reshape(_C, 4).astype(jnp.float32)
    wg = Wgcn.astype(jnp.float32)
    b = bgcn.reshape(_C, 1).astype(jnp.float32)

    out = pl.pallas_call(
        _body,
        grid=(_NSTEPS,),
        in_specs=[
            pl.BlockSpec((2 * _R, _IMG), lambda i: (jnp.maximum(i - 1, 0), 0)),
            pl.BlockSpec((2 * _R, _IMG), lambda i: (i, 0)),
            pl.BlockSpec((2 * _R, _IMG),
                         lambda i: (jnp.minimum(i + 1, _NSTEPS - 1), 0)),
            pl.BlockSpec((2 * _IMG, 4 * _W), lambda i: (0, 0)),
            pl.BlockSpec((_C, 4), lambda i: (0, 0)),
            pl.BlockSpec((_C, _C), lambda i: (0, 0)),
            pl.BlockSpec((_C, 1), lambda i: (0, 0)),
            pl.BlockSpec((1, _R + 2, 4 * _W), lambda i: (i, 0, 0)),
            pl.BlockSpec((1, 4 * _W), lambda i: (0, 0)),
            pl.BlockSpec((1, 4 * _W), lambda i: (0, 0)),
            pl.BlockSpec((1, 1, _R * _W), lambda i: (i, 0, 0)),
        ],
        out_specs=pl.BlockSpec((_C, _R * _W), lambda i: (0, i)),
        out_shape=jax.ShapeDtypeStruct((_C, _N), jnp.float32),
    )(xr, xr, xr, jnp.asarray(_SM), wf, wg, b, jnp.asarray(_DSRC),
      jnp.asarray(_ML), jnp.asarray(_MR), jnp.asarray(_DDST))

    return out.T.reshape(1, _N, _C)
